# baseline (speedup 1.0000x reference)
import jax
import jax.numpy as jnp
from jax import lax
from jax.experimental import pallas as pl
from jax.experimental.pallas import tpu as pltpu
from jax.experimental.pallas import tpu_sc as plsc

L = 200
B = 4096
D = 64
OUT = 2
V = 1_000_000
NC, NS = 2, 16
NW = NC * NS
COLS = B // NW
ROWS = 25
CHUNK = ROWS * COLS
STEPS = L // ROWS
NBUF = 4

_C = 65536

_mesh = plsc.VectorSubcoreMesh(
    core_axis_name="c", subcore_axis_name="s", num_cores=NC, num_subcores=NS
)


def _proj_body(t_ref, w_ref, o0_ref, o1_ref):
    p = jnp.dot(w_ref[...], t_ref[...], preferred_element_type=jnp.float32)
    o0_ref[...] = p[0]
    o1_ref[...] = p[1]


def _project(table_t, w):
    grid = pl.cdiv(V, _C)
    return pl.pallas_call(
        _proj_body,
        grid=(grid,),
        in_specs=[
            pl.BlockSpec((D, _C), lambda i: (0, i)),
            pl.BlockSpec((OUT, D), lambda i: (0, 0)),
        ],
        out_specs=[
            pl.BlockSpec((_C,), lambda i: (i,)),
            pl.BlockSpec((_C,), lambda i: (i,)),
        ],
        out_shape=[jax.ShapeDtypeStruct((V,), jnp.float32)] * 2,
    )(table_t, w)


@jax.jit
def _emb_avg(text, table, w, b):
    table_t = table.T
    p0, p1 = _project(table_t, w)
    b0v = jnp.broadcast_to(b[0], (16,))
    b1v = jnp.broadcast_to(b[1], (16,))
    dummy = jnp.zeros((CHUNK,), jnp.float32)
    text_r = text.reshape(L, NW, COLS).transpose(1, 0, 2).reshape(NW, L * COLS)

    @pl.kernel(
        out_type=jax.ShapeDtypeStruct((B * OUT,), jnp.float32),
        mesh=_mesh,
        scratch_types=[
            pltpu.VMEM((L * COLS,), jnp.int32),
            pltpu.VMEM((NBUF, OUT, CHUNK), jnp.float32),
            pltpu.VMEM((16,), jnp.float32),
            pltpu.VMEM((16,), jnp.float32),
            pltpu.VMEM((COLS * OUT,), jnp.float32),
            pltpu.SemaphoreType.DMA,
        ],
        compiler_params=pltpu.CompilerParams(
            use_tc_tiling_on_sc=False, needs_layout_passes=False
        ),
    )
    def body(text_hbm, p0_hbm, p1_hbm, b0_hbm, b1_hbm, dum_hbm, out_hbm,
             t_v, g_v, b0_v, b1_v, out_v, sem):
        wid = lax.axis_index("s") * NC + lax.axis_index("c")
        base = wid * COLS
        lane = lax.iota(jnp.int32, 16)
        zero = jnp.zeros((16,), jnp.float32)

        pltpu.sync_copy(b0_hbm, b0_v)
        pltpu.sync_copy(b1_hbm, b1_v)
        pltpu.sync_copy(text_hbm.at[wid], t_v)

        def issue(step, pb):
            idx = t_v.at[pl.ds(CHUNK * step, CHUNK)]
            pltpu.async_copy(p0_hbm.at[idx], g_v.at[pb, 0], sem)
            pltpu.async_copy(p1_hbm.at[idx], g_v.at[pb, 1], sem)

        for pb in range(NBUF):
            issue(pb, pb)

        def round_body(i, accs):
            accs = list(accs)
            for j in range(NBUF):
                step = NBUF * i + j
                for s in range(OUT):
                    pltpu.make_async_copy(dum_hbm, g_v.at[j, s], sem).wait()
                for r in range(ROWS):
                    for k in range(8):
                        off = 128 * r + 16 * k
                        accs[k] = accs[k] + g_v[j, 0, pl.ds(off, 16)]
                        accs[8 + k] = accs[8 + k] + g_v[j, 1, pl.ds(off, 16)]

                @pl.when(step + NBUF < STEPS)
                def _():
                    issue(step + NBUF, j)

            return tuple(accs)

        accs = lax.fori_loop(0, STEPS // NBUF, round_body, (zero,) * 16)

        inv = jnp.float32(1.0 / L)
        b0_reg = b0_v[pl.ds(0, 16)]
        b1_reg = b1_v[pl.ds(0, 16)]
        for k in range(8):
            v0 = accs[k] * inv + b0_reg
            v1 = accs[8 + k] * inv + b1_reg
            plsc.store_scatter(out_v, [32 * k + 2 * lane], v0)
            plsc.store_scatter(out_v, [32 * k + 2 * lane + 1], v1)

        pltpu.sync_copy(out_v, out_hbm.at[pl.ds(base * OUT, COLS * OUT)])

    return body(text_r, p0, p1, b0v, b1v, dummy)


def kernel(text, table, W, b):
    return _emb_avg(text, table, W, b).reshape(B, OUT)


# speedup vs baseline: 1.0275x; 1.0275x over previous
import jax
import jax.numpy as jnp
from jax import lax
from jax.experimental import pallas as pl
from jax.experimental.pallas import tpu as pltpu
from jax.experimental.pallas import tpu_sc as plsc

L = 200
B = 4096
D = 64
OUT = 2
V = 1_000_000
NC, NS = 2, 16
NW = NC * NS
COLS = B // NW
ROWS = 25
CHUNK = ROWS * COLS
STEPS = L // ROWS
NBUF = 4

_C = 32768

_mesh = plsc.VectorSubcoreMesh(
    core_axis_name="c", subcore_axis_name="s", num_cores=NC, num_subcores=NS
)


def _proj_body(t_ref, w_ref, o0_ref, o1_ref):
    p = jnp.dot(w_ref[...], t_ref[...], preferred_element_type=jnp.float32)
    o0_ref[...] = p[0]
    o1_ref[...] = p[1]


def _project(table_t, w):
    grid = pl.cdiv(V, _C)
    return pl.pallas_call(
        _proj_body,
        grid=(grid,),
        in_specs=[
            pl.BlockSpec((D, _C), lambda i: (0, i)),
            pl.BlockSpec((OUT, D), lambda i: (0, 0)),
        ],
        out_specs=[
            pl.BlockSpec((_C,), lambda i: (i,)),
            pl.BlockSpec((_C,), lambda i: (i,)),
        ],
        out_shape=[jax.ShapeDtypeStruct((V,), jnp.float32)] * 2,
    )(table_t, w)


@jax.jit
def _emb_avg(text, table, w, b):
    table_t = table.T
    p0, p1 = _project(table_t, w)
    bvec = jnp.tile(b, 16 // OUT)
    text_r = text.reshape(L, NW, COLS).transpose(1, 0, 2).reshape(NW, L * COLS)

    @pl.kernel(
        out_type=jax.ShapeDtypeStruct((B * OUT,), jnp.float32),
        mesh=_mesh,
        scratch_types=[
            pltpu.VMEM((L * COLS,), jnp.int32),
            pltpu.VMEM((NBUF, OUT, CHUNK), jnp.float32),
            pltpu.VMEM((16,), jnp.float32),
            pltpu.VMEM((COLS * OUT,), jnp.float32),
            pltpu.SemaphoreType.DMA,
        ],
        compiler_params=pltpu.CompilerParams(
            use_tc_tiling_on_sc=False, needs_layout_passes=False
        ),
    )
    def body(text_hbm, p0_hbm, p1_hbm, b_hbm, out_hbm,
             t_v, g_v, b_v, out_v, sem):
        wid = lax.axis_index("s") * NC + lax.axis_index("c")
        base = wid * COLS
        lane = lax.iota(jnp.int32, 16)
        zero = jnp.zeros((16,), jnp.float32)

        pltpu.sync_copy(b_hbm, b_v)
        pltpu.sync_copy(text_hbm.at[wid], t_v)

        def issue(step, pb):
            idx = t_v.at[pl.ds(CHUNK * step, CHUNK)]
            pltpu.async_copy(p0_hbm.at[idx], g_v.at[pb, 0], sem)
            pltpu.async_copy(p1_hbm.at[idx], g_v.at[pb, 1], sem)

        for pb in range(NBUF):
            issue(pb, pb)

        def round_body(i, accs):
            accs = list(accs)
            for j in range(NBUF):
                step = NBUF * i + j
                for s in range(OUT):
                    pltpu.make_async_copy(
                        p0_hbm.at[pl.ds(0, CHUNK)], g_v.at[j, s], sem
                    ).wait()
                for r in range(ROWS):
                    for k in range(8):
                        off = 128 * r + 16 * k
                        accs[k] = accs[k] + g_v[j, 0, pl.ds(off, 16)]
                        accs[8 + k] = accs[8 + k] + g_v[j, 1, pl.ds(off, 16)]

                @pl.when(step + NBUF < STEPS)
                def _():
                    issue(step + NBUF, j)

            return tuple(accs)

        accs = lax.fori_loop(0, STEPS // NBUF, round_body, (zero,) * 16)

        inv = jnp.float32(1.0 / L)
        b0_reg = plsc.load_gather(b_v, [jnp.zeros((16,), jnp.int32)])
        b1_reg = plsc.load_gather(b_v, [jnp.ones((16,), jnp.int32)])
        for k in range(8):
            v0 = accs[k] * inv + b0_reg
            v1 = accs[8 + k] * inv + b1_reg
            plsc.store_scatter(out_v, [32 * k + 2 * lane], v0)
            plsc.store_scatter(out_v, [32 * k + 2 * lane + 1], v1)

        pltpu.sync_copy(out_v, out_hbm.at[pl.ds(base * OUT, COLS * OUT)])

    return body(text_r, p0, p1, bvec)


def kernel(text, table, W, b):
    return _emb_avg(text, table, W, b).reshape(B, OUT)
